# Initial kernel scaffold; baseline (speedup 1.0000x reference)
import jax
import jax.numpy as jnp
from jax.experimental import pallas as pl


def kernel(X, edge_index, W, a_self, a_neigh, bias):
    raise NotImplementedError("write your pallas kernel here")


# speedup vs baseline: 17.7718x; 17.7718x over previous
import functools

import jax
import jax.numpy as jnp
from jax import lax
from jax.experimental import pallas as pl
from jax.experimental.pallas import tpu as pltpu
from jax.experimental.pallas import tpu_sc as plsc

_N = 10000
_F = 128
_FX = 144
_NC = 2
_NS = 16
_NW = _NC * _NS
_C = 128
_NPT = _N // _NS


def _prep_body(x_ref, w_ref, asw_ref, anw_ref, featx_ref, asv_ref, anv_ref):
    feat = jnp.dot(x_ref[...], w_ref[...], preferred_element_type=jnp.float32)
    asv = jnp.dot(feat, asw_ref[...], preferred_element_type=jnp.float32)
    anv = jnp.dot(feat, anw_ref[...], preferred_element_type=jnp.float32)
    n = feat.shape[0]
    ext = jnp.concatenate(
        [feat, jnp.ones((n, 1), jnp.float32),
         jnp.zeros((n, _FX - _F - 1), jnp.float32)], axis=1)
    featx_ref[...] = ext
    asv_ref[...] = asv
    anv_ref[...] = anv


def _fin_body(p_ref, b_ref, o_ref):
    t = p_ref[0] + p_ref[1]
    numer = t[:, :_F]
    den = t[:, _F:_F + 1]
    o_ref[...] = jnp.maximum(numer / (den + 1e-9) + b_ref[...], 0.0)


def _make_agg(e_real, e_pad):
    ept = e_pad // _NW
    nchunk = ept // _C
    mesh = plsc.VectorSubcoreMesh(core_axis_name="c", subcore_axis_name="s",
                                  num_cores=_NC)

    @functools.partial(
        pl.kernel,
        out_type=jax.ShapeDtypeStruct((_NC, _N, _FX), jnp.float32),
        mesh=mesh,
        compiler_params=pltpu.CompilerParams(use_tc_tiling_on_sc=False,
                                             needs_layout_passes=False),
        scratch_types=[
            pltpu.VMEM((_N,), jnp.float32),
            pltpu.VMEM((_N,), jnp.float32),
            pltpu.VMEM((_C, _FX), jnp.float32),
            pltpu.VMEM((_C,), jnp.float32),
            pltpu.VMEM((_C,), jnp.int32),
            pltpu.VMEM((_C,), jnp.int32),
            pltpu.VMEM_SHARED((_N, _FX), jnp.float32),
            pltpu.SemaphoreType.DMA,
        ])
    def agg(rows_hbm, cols_hbm, asv_hbm, anv_hbm, featx_hbm, out_hbm,
            asv_v, anv_v, fbuf, wbuf, cidx, ridx, acc, sem):
        c = lax.axis_index("c")
        s = lax.axis_index("s")
        tid = c * _NS + s
        base = tid * ept

        pltpu.sync_copy(asv_hbm, asv_v)
        pltpu.sync_copy(anv_hbm, anv_v)

        def mx(i, m):
            return jnp.maximum(m, anv_v[pl.ds(i * 16, 16)])

        mvec = lax.fori_loop(0, _N // 16, mx, anv_v[pl.ds(0, 16)])
        mn = lax.reduce_max(mvec, (0,))

        zeros16 = jnp.zeros((16,), jnp.float32)

        def zrow(r, _):
            for k in range(_FX // 16):
                fbuf[r, pl.ds(k * 16, 16)] = zeros16
            return 0

        lax.fori_loop(0, _C, zrow, 0)
        nbase = s * _NPT
        for k in range(_NPT // 125):
            pltpu.sync_copy(fbuf.at[pl.ds(0, 125), :],
                            acc.at[pl.ds(nbase + k * 125, 125), :])
        plsc.subcore_barrier()

        def chunk_body(g, _):
            cb = base + g * _C
            pltpu.sync_copy(cols_hbm.at[pl.ds(cb, _C)], cidx)
            pltpu.sync_copy(rows_hbm.at[pl.ds(cb, _C)], ridx)
            pltpu.async_copy(featx_hbm.at[cidx], fbuf, sem).wait()
            for j in range(_C // 16):
                rv = ridx[pl.ds(j * 16, 16)]
                cv = cidx[pl.ds(j * 16, 16)]
                ar = plsc.load_gather(asv_v, [rv])
                ac = plsc.load_gather(anv_v, [cv])
                z = ar + ac
                e = jnp.where(z > 0, z, 0.2 * z)
                num = jnp.exp(e - jnp.maximum(ar + mn, 0.0))
                gid = cb + j * 16 + lax.iota(jnp.int32, 16)
                num = jnp.where(gid < e_real, num, 0.0)
                wbuf[pl.ds(j * 16, 16)] = num

            def scale_body(i, _):
                wv = plsc.load_gather(wbuf, [lax.broadcast(i, (16,))])
                for k in range(_FX // 16):
                    fbuf[i, pl.ds(k * 16, 16)] = (
                        fbuf[i, pl.ds(k * 16, 16)] * wv)
                return 0

            lax.fori_loop(0, _C, scale_body, 0)
            pltpu.sync_copy(fbuf, acc.at[ridx], add=True)
            return 0

        lax.fori_loop(0, nchunk, chunk_body, 0)
        plsc.subcore_barrier()
        pltpu.sync_copy(acc.at[pl.ds(nbase, _NPT), :],
                        out_hbm.at[c, pl.ds(nbase, _NPT), :])

    return agg


def kernel(X, edge_index, W, a_self, a_neigh, bias):
    e_real = edge_index.shape[1]
    e_pad = ((e_real + _NW * _C - 1) // (_NW * _C)) * (_NW * _C)
    row = edge_index[0]
    col = edge_index[1]
    if e_pad != e_real:
        pad = e_pad - e_real
        row = jnp.concatenate([row, jnp.zeros((pad,), row.dtype)])
        col = jnp.concatenate([col, jnp.zeros((pad,), col.dtype)])

    featx, asv, anv = pl.pallas_call(
        _prep_body,
        out_shape=[
            jax.ShapeDtypeStruct((_N, _FX), jnp.float32),
            jax.ShapeDtypeStruct((_N, 1), jnp.float32),
            jax.ShapeDtypeStruct((_N, 1), jnp.float32),
        ],
    )(X, W, a_self, a_neigh)

    agg = _make_agg(e_real, e_pad)
    partials = agg(row, col, asv.reshape(-1), anv.reshape(-1), featx)

    out = pl.pallas_call(
        _fin_body,
        out_shape=jax.ShapeDtypeStruct((_N, _F), jnp.float32),
    )(partials, bias.reshape(1, _F))
    return out


# speedup vs baseline: 24.6756x; 1.3885x over previous
import functools

import jax
import jax.numpy as jnp
from jax import lax
from jax.experimental import pallas as pl
from jax.experimental.pallas import tpu as pltpu
from jax.experimental.pallas import tpu_sc as plsc

_N = 10000
_F = 128
_FX = 144
_NC = 2
_NS = 16
_NW = _NC * _NS
_C = 64
_NPT = _N // _NS

_DNUMS = lax.GatherDimensionNumbers(
    offset_dims=(), collapsed_slice_dims=(0,), start_index_map=(0,))


def _lane(vec, i):
    idx = jnp.full((16, 1), i, jnp.int32)
    return lax.gather(vec, idx, _DNUMS, (1,),
                      mode=lax.GatherScatterMode.PROMISE_IN_BOUNDS)


def _prep_body(x_ref, w_ref, asw_ref, anw_ref, featx_ref, asv_ref, anv_ref):
    feat = jnp.dot(x_ref[...], w_ref[...], preferred_element_type=jnp.float32)
    asv = jnp.dot(feat, asw_ref[...], preferred_element_type=jnp.float32)
    anv = jnp.dot(feat, anw_ref[...], preferred_element_type=jnp.float32)
    n = feat.shape[0]
    ext = jnp.concatenate(
        [feat, jnp.ones((n, 1), jnp.float32),
         jnp.zeros((n, _FX - _F - 1), jnp.float32)], axis=1)
    featx_ref[...] = ext
    asv_ref[...] = asv
    anv_ref[...] = anv


def _fin_body(p_ref, b_ref, o_ref):
    t = p_ref[0] + p_ref[1]
    numer = t[:, :_F]
    den = t[:, _F:_F + 1]
    o_ref[...] = jnp.maximum(numer / (den + 1e-9) + b_ref[...], 0.0)


def _make_agg(e_real, e_pad):
    ept = e_pad // _NW
    nchunk = ept // _C
    mesh = plsc.VectorSubcoreMesh(core_axis_name="c", subcore_axis_name="s",
                                  num_cores=_NC)

    @functools.partial(
        pl.kernel,
        out_type=jax.ShapeDtypeStruct((_NC, _N, _FX), jnp.float32),
        mesh=mesh,
        compiler_params=pltpu.CompilerParams(use_tc_tiling_on_sc=False,
                                             needs_layout_passes=False),
        scratch_types=[
            pltpu.VMEM((2, _C), jnp.int32),
            pltpu.VMEM((2, _C), jnp.int32),
            pltpu.VMEM((_C, _FX), jnp.float32),
            pltpu.VMEM((_C, _FX), jnp.float32),
            pltpu.VMEM((_N,), jnp.float32),
            pltpu.VMEM((_N,), jnp.float32),
            pltpu.VMEM_SHARED((_N, _FX), jnp.float32),
            pltpu.SemaphoreType.DMA,
            pltpu.SemaphoreType.DMA,
            pltpu.SemaphoreType.DMA,
            pltpu.SemaphoreType.DMA,
        ])
    def agg(edge_hbm, asv_hbm, anv_hbm, featx_hbm, out_hbm,
            ebuf0, ebuf1, fbuf0, fbuf1, asv_v, anv_v, acc,
            semg0, semg1, semi0, semi1):
        c = lax.axis_index("c")
        s = lax.axis_index("s")
        tid = c * _NS + s
        base = tid * ept

        pltpu.sync_copy(asv_hbm, asv_v)
        pltpu.sync_copy(anv_hbm, anv_v)

        def mx(i, m):
            return jnp.maximum(m, anv_v[pl.ds(i * 16, 16)])

        mvec = lax.fori_loop(0, _N // 16, mx, anv_v[pl.ds(0, 16)])
        mn = lax.reduce_max(mvec, (0,))

        zeros16 = jnp.zeros((16,), jnp.float32)

        def zrow(r, _):
            for k in range(_FX // 16):
                fbuf0[r, pl.ds(k * 16, 16)] = zeros16
            return 0

        lax.fori_loop(0, _C, zrow, 0)
        nbase = s * _NPT
        for k in range(_NPT // _C):
            pltpu.sync_copy(fbuf0.at[pl.ds(0, _C), :],
                            acc.at[pl.ds(nbase + k * _C, _C), :])
        rem = _NPT % _C
        if rem:
            pltpu.sync_copy(fbuf0.at[pl.ds(0, rem), :],
                            acc.at[pl.ds(nbase + _NPT - rem, rem), :])
        plsc.subcore_barrier()

        pltpu.sync_copy(edge_hbm.at[:, pl.ds(base, _C)], ebuf0)
        pltpu.async_copy(featx_hbm.at[ebuf0.at[1]], fbuf0, semg0)
        pltpu.async_copy(edge_hbm.at[:, pl.ds(base + _C, _C)], ebuf1, semi1)

        def halfstep(g, ebuf_p, fbuf_p, semg_p, semi_p, ebuf_q, fbuf_q,
                     semg_q, semi_q):
            cb = base + g * _C
            pltpu.make_async_copy(featx_hbm.at[ebuf_p.at[1]], fbuf_p,
                                  semg_p).wait()

            @pl.when(g + 1 < nchunk)
            def _prefetch_gather():
                pltpu.make_async_copy(
                    edge_hbm.at[:, pl.ds(cb + _C, _C)], ebuf_q, semi_q).wait()
                pltpu.async_copy(featx_hbm.at[ebuf_q.at[1]], fbuf_q, semg_q)

            def group(j, _):
                jj = j * 16
                rv = ebuf_p[0, pl.ds(jj, 16)]
                cv = ebuf_p[1, pl.ds(jj, 16)]
                ar = plsc.load_gather(asv_v, [rv])
                ac = plsc.load_gather(anv_v, [cv])
                z = ar + ac
                e = jnp.where(z > 0, z, 0.2 * z)
                num = jnp.exp(e - jnp.maximum(ar + mn, 0.0))
                gid = cb + jj + lax.iota(jnp.int32, 16)
                num = jnp.where(gid < e_real, num, 0.0)
                for el in range(16):
                    wv = _lane(num, el)
                    row = jj + el
                    for k in range(_FX // 16):
                        fbuf_p[row, pl.ds(k * 16, 16)] = (
                            fbuf_p[row, pl.ds(k * 16, 16)] * wv)
                return 0

            lax.fori_loop(0, _C // 16, group, 0)
            pltpu.sync_copy(fbuf_p, acc.at[ebuf_p.at[0]], add=True)

            @pl.when(g + 2 < nchunk)
            def _prefetch_idx():
                pltpu.async_copy(edge_hbm.at[:, pl.ds(cb + 2 * _C, _C)],
                                 ebuf_p, semi_p)

        def body(g2, _):
            g = g2 * 2
            halfstep(g, ebuf0, fbuf0, semg0, semi0, ebuf1, fbuf1, semg1,
                     semi1)
            halfstep(g + 1, ebuf1, fbuf1, semg1, semi1, ebuf0, fbuf0, semg0,
                     semi0)
            return 0

        lax.fori_loop(0, nchunk // 2, body, 0)
        plsc.subcore_barrier()
        pltpu.sync_copy(acc.at[pl.ds(nbase, _NPT), :],
                        out_hbm.at[c, pl.ds(nbase, _NPT), :])

    return agg


def kernel(X, edge_index, W, a_self, a_neigh, bias):
    e_real = edge_index.shape[1]
    quantum = _NW * _C * 2
    e_pad = ((e_real + quantum - 1) // quantum) * quantum
    edges = edge_index
    if e_pad != e_real:
        edges = jnp.pad(edge_index, ((0, 0), (0, e_pad - e_real)))

    featx, asv, anv = pl.pallas_call(
        _prep_body,
        out_shape=[
            jax.ShapeDtypeStruct((_N, _FX), jnp.float32),
            jax.ShapeDtypeStruct((_N, 1), jnp.float32),
            jax.ShapeDtypeStruct((_N, 1), jnp.float32),
        ],
    )(X, W, a_self, a_neigh)

    agg = _make_agg(e_real, e_pad)
    partials = agg(edges, asv.reshape(-1), anv.reshape(-1), featx)

    out = pl.pallas_call(
        _fin_body,
        out_shape=jax.ShapeDtypeStruct((_N, _F), jnp.float32),
    )(partials, bias.reshape(1, _F))
    return out


# speedup vs baseline: 25.9954x; 1.0535x over previous
import functools

import jax
import jax.numpy as jnp
from jax import lax
from jax.experimental import pallas as pl
from jax.experimental.pallas import tpu as pltpu
from jax.experimental.pallas import tpu_sc as plsc

_N = 10000
_F = 128
_FX = 144
_NC = 2
_NS = 16
_NW = _NC * _NS
_C = 64
_NPT = _N // _NS

_DNUMS = lax.GatherDimensionNumbers(
    offset_dims=(), collapsed_slice_dims=(0,), start_index_map=(0,))


def _lane(vec, i):
    idx = jnp.full((16, 1), i, jnp.int32)
    return lax.gather(vec, idx, _DNUMS, (1,),
                      mode=lax.GatherScatterMode.PROMISE_IN_BOUNDS)


def _prep_body(x_ref, w_ref, asw_ref, anw_ref, featx_ref, asv_ref, anv_ref):
    feat = jnp.dot(x_ref[...], w_ref[...], preferred_element_type=jnp.float32)
    asv = jnp.dot(feat, asw_ref[...], preferred_element_type=jnp.float32)
    anv = jnp.dot(feat, anw_ref[...], preferred_element_type=jnp.float32)
    n = feat.shape[0]
    ext = jnp.concatenate(
        [feat, jnp.ones((n, 1), jnp.float32),
         jnp.zeros((n, _FX - _F - 1), jnp.float32)], axis=1)
    featx_ref[...] = ext
    asv_ref[...] = asv
    anv_ref[...] = anv


def _fin_body(p_ref, b_ref, o_ref):
    t = p_ref[0] + p_ref[1]
    numer = t[:, :_F]
    den = t[:, _F:_F + 1]
    o_ref[...] = jnp.maximum(numer / (den + 1e-9) + b_ref[...], 0.0)


def _make_agg(e_real, e_pad):
    ept = e_pad // _NW
    nchunk = ept // _C
    mesh = plsc.VectorSubcoreMesh(core_axis_name="c", subcore_axis_name="s",
                                  num_cores=_NC)

    @functools.partial(
        pl.kernel,
        out_type=jax.ShapeDtypeStruct((_NC, _N, _FX), jnp.float32),
        mesh=mesh,
        compiler_params=pltpu.CompilerParams(use_tc_tiling_on_sc=False,
                                             needs_layout_passes=False),
        scratch_types=[
            pltpu.VMEM((2, _C), jnp.int32),
            pltpu.VMEM((2, _C), jnp.int32),
            pltpu.VMEM((_C, _FX), jnp.float32),
            pltpu.VMEM((_C, _FX), jnp.float32),
            pltpu.VMEM((_C,), jnp.int32),
            pltpu.VMEM((_C,), jnp.int32),
            pltpu.VMEM((_N,), jnp.float32),
            pltpu.VMEM((_N,), jnp.float32),
            pltpu.VMEM_SHARED((_N, _FX), jnp.float32),
            pltpu.SemaphoreType.DMA,
            pltpu.SemaphoreType.DMA,
            pltpu.SemaphoreType.DMA,
            pltpu.SemaphoreType.DMA,
            pltpu.SemaphoreType.DMA,
            pltpu.SemaphoreType.DMA,
        ])
    def agg(edge_hbm, asv_hbm, anv_hbm, featx_hbm, out_hbm,
            ebuf0, ebuf1, fbuf0, fbuf1, rbuf0, rbuf1, asv_v, anv_v, acc,
            semg0, semg1, semi0, semi1, sems0, sems1):
        c = lax.axis_index("c")
        s = lax.axis_index("s")
        tid = c * _NS + s
        base = tid * ept

        pltpu.sync_copy(asv_hbm, asv_v)
        pltpu.sync_copy(anv_hbm, anv_v)

        def mx(i, m):
            return jnp.maximum(m, anv_v[pl.ds(i * 16, 16)])

        mvec = lax.fori_loop(0, _N // 16, mx, anv_v[pl.ds(0, 16)])
        mn = lax.reduce_max(mvec, (0,))

        zeros16 = jnp.zeros((16,), jnp.float32)

        def zrow(r, _):
            for k in range(_FX // 16):
                fbuf0[r, pl.ds(k * 16, 16)] = zeros16
            return 0

        lax.fori_loop(0, _C, zrow, 0)
        nbase = s * _NPT
        for k in range(_NPT // _C):
            pltpu.sync_copy(fbuf0.at[pl.ds(0, _C), :],
                            acc.at[pl.ds(nbase + k * _C, _C), :])
        rem = _NPT % _C
        if rem:
            pltpu.sync_copy(fbuf0.at[pl.ds(0, rem), :],
                            acc.at[pl.ds(nbase + _NPT - rem, rem), :])
        plsc.subcore_barrier()

        pltpu.sync_copy(edge_hbm.at[:, pl.ds(base, _C)], ebuf0)
        pltpu.async_copy(featx_hbm.at[ebuf0.at[1]], fbuf0, semg0)
        pltpu.async_copy(edge_hbm.at[:, pl.ds(base + _C, _C)], ebuf1, semi1)

        def halfstep(g, ebuf_p, fbuf_p, rbuf_p, semg_p, semi_p, sems_p,
                     ebuf_q, fbuf_q, rbuf_q, semg_q, semi_q, sems_q):
            cb = base + g * _C
            pltpu.make_async_copy(featx_hbm.at[ebuf_p.at[1]], fbuf_p,
                                  semg_p).wait()

            @pl.when(g + 1 < nchunk)
            def _prefetch_gather():
                pltpu.make_async_copy(
                    edge_hbm.at[:, pl.ds(cb + _C, _C)], ebuf_q, semi_q).wait()

                @pl.when(g > 0)
                def _wait_scatter():
                    pltpu.make_async_copy(fbuf_q, acc.at[rbuf_q],
                                          sems_q).wait()

                pltpu.async_copy(featx_hbm.at[ebuf_q.at[1]], fbuf_q, semg_q)

            def group(j, _):
                jj = j * 16
                rv = ebuf_p[0, pl.ds(jj, 16)]
                cv = ebuf_p[1, pl.ds(jj, 16)]
                ar = plsc.load_gather(asv_v, [rv])
                ac = plsc.load_gather(anv_v, [cv])
                z = ar + ac
                e = jnp.where(z > 0, z, 0.2 * z)
                num = jnp.exp(e - jnp.maximum(ar + mn, 0.0))
                gid = cb + jj + lax.iota(jnp.int32, 16)
                num = jnp.where(gid < e_real, num, 0.0)
                for el in range(16):
                    wv = _lane(num, el)
                    row = jj + el
                    for k in range(_FX // 16):
                        fbuf_p[row, pl.ds(k * 16, 16)] = (
                            fbuf_p[row, pl.ds(k * 16, 16)] * wv)
                return 0

            lax.fori_loop(0, _C // 16, group, 0)
            for k in range(_C // 16):
                rbuf_p[pl.ds(k * 16, 16)] = ebuf_p[0, pl.ds(k * 16, 16)]
            pltpu.async_copy(fbuf_p, acc.at[rbuf_p], sems_p, add=True)

            @pl.when(g + 2 < nchunk)
            def _prefetch_idx():
                pltpu.async_copy(edge_hbm.at[:, pl.ds(cb + 2 * _C, _C)],
                                 ebuf_p, semi_p)

        def body(g2, _):
            g = g2 * 2
            halfstep(g, ebuf0, fbuf0, rbuf0, semg0, semi0, sems0,
                     ebuf1, fbuf1, rbuf1, semg1, semi1, sems1)
            halfstep(g + 1, ebuf1, fbuf1, rbuf1, semg1, semi1, sems1,
                     ebuf0, fbuf0, rbuf0, semg0, semi0, sems0)
            return 0

        lax.fori_loop(0, nchunk // 2, body, 0)
        pltpu.make_async_copy(fbuf0, acc.at[rbuf0], sems0).wait()
        pltpu.make_async_copy(fbuf1, acc.at[rbuf1], sems1).wait()
        plsc.subcore_barrier()
        pltpu.sync_copy(acc.at[pl.ds(nbase, _NPT), :],
                        out_hbm.at[c, pl.ds(nbase, _NPT), :])

    return agg


def kernel(X, edge_index, W, a_self, a_neigh, bias):
    e_real = edge_index.shape[1]
    quantum = _NW * _C * 2
    e_pad = ((e_real + quantum - 1) // quantum) * quantum
    edges = edge_index
    if e_pad != e_real:
        edges = jnp.pad(edge_index, ((0, 0), (0, e_pad - e_real)))

    featx, asv, anv = pl.pallas_call(
        _prep_body,
        out_shape=[
            jax.ShapeDtypeStruct((_N, _FX), jnp.float32),
            jax.ShapeDtypeStruct((_N, 1), jnp.float32),
            jax.ShapeDtypeStruct((_N, 1), jnp.float32),
        ],
    )(X, W, a_self, a_neigh)

    agg = _make_agg(e_real, e_pad)
    partials = agg(edges, asv.reshape(-1), anv.reshape(-1), featx)

    out = pl.pallas_call(
        _fin_body,
        out_shape=jax.ShapeDtypeStruct((_N, _F), jnp.float32),
    )(partials, bias.reshape(1, _F))
    return out
